# baseline (speedup 1.0000x reference)
import functools
import math

import jax
import jax.numpy as jnp
from jax import lax
from jax.experimental import pallas as pl
from jax.experimental.pallas import tpu as pltpu

N_NODES = 50000
D_FEAT = 128
ROW_BLK = 1000


def _dense_stage_body(a1_ref, a2_ref, w1_ref, w2_ref, b1_ref, b2_ref, o_ref):
    h1 = jnp.dot(a1_ref[...], w1_ref[...], preferred_element_type=jnp.float32)
    h2 = jnp.dot(a2_ref[...], w2_ref[...], preferred_element_type=jnp.float32)
    r1 = jnp.maximum(h1 + b1_ref[...], 0.0)
    r2 = jnp.maximum(h2 + b2_ref[...], 0.0)
    o_ref[...] = (r1 + r2) * 0.5


def _dense_stage(a1, a2, W1, W2, b1, b2):
    n = a1.shape[0]
    grid = (n // ROW_BLK,)
    blk = lambda i: (i, 0)
    full = lambda i: (0, 0)
    return pl.pallas_call(
        _dense_stage_body,
        grid=grid,
        in_specs=[
            pl.BlockSpec((ROW_BLK, D_FEAT), blk),
            pl.BlockSpec((ROW_BLK, D_FEAT), blk),
            pl.BlockSpec((D_FEAT, D_FEAT), full),
            pl.BlockSpec((D_FEAT, D_FEAT), full),
            pl.BlockSpec((1, D_FEAT), full),
            pl.BlockSpec((1, D_FEAT), full),
        ],
        out_specs=pl.BlockSpec((ROW_BLK, D_FEAT), blk),
        out_shape=jax.ShapeDtypeStruct((n, D_FEAT), jnp.float32),
    )(a1, a2, W1, W2, b1.reshape(1, -1), b2.reshape(1, -1))


def _edge_norm_both(val, src, dst, n_src, n_dst):
    deg_src = jnp.zeros((n_src,), val.dtype).at[src].add(val)
    deg_dst = jnp.zeros((n_dst,), val.dtype).at[dst].add(val)
    return val / jnp.sqrt(
        jnp.maximum(deg_src[src], 1e-12) * jnp.maximum(deg_dst[dst], 1e-12))


def _agg_raw(x_src, src, dst, w, n_dst):
    return jnp.zeros((n_dst, x_src.shape[1]), x_src.dtype).at[dst].add(
        x_src[src] * w[:, None])


def kernel(com_emb, pos_emb, demand_edge_index, supply_edge_index,
           comflow_edge_index, posflow_edge_index,
           demand_val, supply_val, comflow_val, posflow_val,
           W_demand, b_demand, W_supply, b_supply,
           W_comflow, b_comflow, W_posflow, b_posflow):
    d_src, d_dst = demand_edge_index[0], demand_edge_index[1]
    s_src, s_dst = supply_edge_index[0], supply_edge_index[1]
    c_src, c_dst = comflow_edge_index[0], comflow_edge_index[1]
    p_src, p_dst = posflow_edge_index[0], posflow_edge_index[1]

    wd = _edge_norm_both(demand_val, d_src, d_dst, N_NODES, N_NODES)
    ws = _edge_norm_both(supply_val, s_src, s_dst, N_NODES, N_NODES)
    wc = _edge_norm_both(comflow_val, c_src, c_dst, N_NODES, N_NODES)
    wp = _edge_norm_both(posflow_val, p_src, p_dst, N_NODES, N_NODES)

    agg_d = _agg_raw(com_emb, d_src, d_dst, wd, N_NODES)
    agg_s = _agg_raw(pos_emb, s_src, s_dst, ws, N_NODES)
    agg_c = _agg_raw(com_emb, c_src, c_dst, wc, N_NODES)
    agg_p = _agg_raw(pos_emb, p_src, p_dst, wp, N_NODES)

    com_out = _dense_stage(agg_s, agg_c, W_supply, W_comflow, b_supply, b_comflow)
    pos_out = _dense_stage(agg_d, agg_p, W_demand, W_posflow, b_demand, b_posflow)
    return (com_out, pos_out)
